# baseline (speedup 1.0000x reference)
import functools

import jax
import jax.numpy as jnp
from jax import lax
from jax.experimental import pallas as pl
from jax.experimental.pallas import tpu as pltpu
from jax.experimental.pallas import tpu_sc as plsc

_EPS = 1e-5
_S_BLK = 2048


def _ln_body(x_ref, pos_ref, g_ref, b_ref, o_ref):
    x = x_ref[...]
    p = pos_ref[...]
    e = x + p[None, :, :]
    mean = jnp.mean(e, axis=-1, keepdims=True)
    c = e - mean
    var = jnp.mean(c * c, axis=-1, keepdims=True)
    o_ref[...] = c * jax.lax.rsqrt(var + _EPS) * g_ref[...] + b_ref[...]


def _rsqrt_newton(v):
    i = plsc.bitcast(v, jnp.int32)
    i = jnp.int32(0x5F3759DF) - (i >> 1)
    y = plsc.bitcast(i, jnp.float32)
    for _ in range(3):
        y = y * (1.5 - 0.5 * v * y * y)
    return y


def _sc_layernorm(input_feat, pos_table, ln_gamma, ln_beta):
    bsz, seq, d = input_feat.shape
    info = plsc.get_sparse_core_info()
    nw = info.num_cores * info.num_subcores
    lanes = info.num_lanes
    span = seq // nw
    r = lanes
    nchunk = span // r
    mesh = plsc.VectorSubcoreMesh(core_axis_name="c", subcore_axis_name="s")

    @functools.partial(
        pl.kernel,
        mesh=mesh,
        out_type=jax.ShapeDtypeStruct((bsz, seq, d), jnp.float32),
        scratch_types=[
            pltpu.VMEM((r, d), jnp.float32),
            pltpu.VMEM((r, d), jnp.float32),
            pltpu.VMEM((r, d), jnp.float32),
            pltpu.VMEM((d,), jnp.float32),
            pltpu.VMEM((d,), jnp.float32),
        ],
        compiler_params=pltpu.CompilerParams(
            use_tc_tiling_on_sc=False, needs_layout_passes=False
        ),
    )
    def k(x_hbm, pos_hbm, g_hbm, b_hbm, out_hbm, xbuf, posbuf, ebuf, gbuf, bbuf):
        wid = lax.axis_index("s") * info.num_cores + lax.axis_index("c")
        s0 = wid * span
        rows = lax.iota(jnp.int32, lanes)
        pltpu.sync_copy(g_hbm, gbuf)
        pltpu.sync_copy(b_hbm, bbuf)

        def chunk_body(ci, _):
            spos = s0 + ci * r
            pltpu.sync_copy(pos_hbm.at[pl.ds(spos, r)], posbuf)
            for b in range(bsz):
                pltpu.sync_copy(x_hbm.at[b, pl.ds(spos, r)], xbuf)

                def colA(j, carry):
                    s, q = carry
                    jv = jnp.full((lanes,), j, jnp.int32)
                    e = plsc.load_gather(xbuf, [rows, jv]) + plsc.load_gather(posbuf, [rows, jv])
                    plsc.store_scatter(ebuf, [rows, jv], e)
                    return s + e, q + e * e

                zero = jnp.zeros((lanes,), jnp.float32)
                s, q = lax.fori_loop(0, d, colA, (zero, zero))
                mean = s * (1.0 / d)
                var = q * (1.0 / d) - mean * mean
                rstd = _rsqrt_newton(var + _EPS)

                def colB(j, _):
                    jv = jnp.full((lanes,), j, jnp.int32)
                    e = plsc.load_gather(ebuf, [rows, jv])
                    o = (e - mean) * rstd * plsc.load_gather(gbuf, [jv]) + plsc.load_gather(bbuf, [jv])
                    plsc.store_scatter(ebuf, [rows, jv], o)
                    return 0

                lax.fori_loop(0, d, colB, 0)
                pltpu.sync_copy(ebuf, out_hbm.at[b, pl.ds(spos, r)])
            return 0

        lax.fori_loop(0, nchunk, chunk_body, 0)

    return k(input_feat, pos_table, ln_gamma, ln_beta)


def _tc_layernorm(input_feat, pos_table, ln_gamma, ln_beta):
    bsz, seq, d = input_feat.shape
    n_s = seq // _S_BLK
    grid = (n_s, bsz)
    return pl.pallas_call(
        _ln_body,
        grid=grid,
        in_specs=[
            pl.BlockSpec((1, _S_BLK, d), lambda i, j: (j, i, 0)),
            pl.BlockSpec((_S_BLK, d), lambda i, j: (i, 0)),
            pl.BlockSpec((d,), lambda i, j: (0,)),
            pl.BlockSpec((d,), lambda i, j: (0,)),
        ],
        out_specs=pl.BlockSpec((1, _S_BLK, d), lambda i, j: (j, i, 0)),
        out_shape=jax.ShapeDtypeStruct((bsz, seq, d), input_feat.dtype),
        compiler_params=pltpu.CompilerParams(
            dimension_semantics=("arbitrary", "arbitrary"),
        ),
    )(input_feat, pos_table, ln_gamma, ln_beta)


def kernel(input_feat, pos_table, ln_gamma, ln_beta):
    return _sc_layernorm(input_feat, pos_table, ln_gamma, ln_beta)


# speedup vs baseline: 3.9729x; 3.9729x over previous
import functools

import jax
import jax.numpy as jnp
from jax import lax
from jax.experimental import pallas as pl
from jax.experimental.pallas import tpu as pltpu
from jax.experimental.pallas import tpu_sc as plsc

_EPS = 1e-5
_S_BLK = 2048


def _ln_body(x_ref, pos_ref, g_ref, b_ref, o_ref):
    x = x_ref[...]
    p = pos_ref[...]
    e = x + p[None, :, :]
    mean = jnp.mean(e, axis=-1, keepdims=True)
    c = e - mean
    var = jnp.mean(c * c, axis=-1, keepdims=True)
    o_ref[...] = c * jax.lax.rsqrt(var + _EPS) * g_ref[...] + b_ref[...]


def _rsqrt_newton(v):
    i = plsc.bitcast(v, jnp.int32)
    i = jnp.int32(0x5F3759DF) - (i >> 1)
    y = plsc.bitcast(i, jnp.float32)
    for _ in range(3):
        y = y * (1.5 - 0.5 * v * y * y)
    return y


def _sc_layernorm(input_feat, pos_table, ln_gamma, ln_beta):
    bsz, seq, d = input_feat.shape
    info = plsc.get_sparse_core_info()
    nw = info.num_cores * info.num_subcores
    lanes = info.num_lanes
    span = seq // nw
    r = lanes
    nchunk = span // r
    nvec = d // lanes
    unroll = 8
    mesh = plsc.VectorSubcoreMesh(core_axis_name="c", subcore_axis_name="s")

    @functools.partial(
        pl.kernel,
        mesh=mesh,
        out_type=jax.ShapeDtypeStruct((bsz, seq, d), jnp.float32),
        scratch_types=[
            pltpu.VMEM((r, d), jnp.float32),
            pltpu.VMEM((r, d), jnp.float32),
            pltpu.VMEM((r, d), jnp.float32),
            pltpu.VMEM((d,), jnp.float32),
            pltpu.VMEM((d,), jnp.float32),
        ],
        compiler_params=pltpu.CompilerParams(
            use_tc_tiling_on_sc=False, needs_layout_passes=False
        ),
    )
    def k(x_hbm, pos_hbm, g_hbm, b_hbm, out_hbm, xbuf, posbuf, ebuf, gbuf, bbuf):
        wid = lax.axis_index("s") * info.num_cores + lax.axis_index("c")
        s0 = wid * span
        pltpu.sync_copy(g_hbm, gbuf)
        pltpu.sync_copy(b_hbm, bbuf)

        def chunk_body(ci, _):
            spos = s0 + ci * r
            pltpu.sync_copy(pos_hbm.at[pl.ds(spos, r)], posbuf)
            for b in range(bsz):
                pltpu.sync_copy(x_hbm.at[b, pl.ds(spos, r)], xbuf)

                def row_body(rw, _):
                    xr, pr, er = xbuf.at[rw], posbuf.at[rw], ebuf.at[rw]
                    zero = jnp.zeros((lanes,), jnp.float32)
                    s = [zero] * 4
                    q = [zero] * 4
                    for v in range(nvec):
                        off = v * lanes
                        e = xr[pl.ds(off, lanes)] + pr[pl.ds(off, lanes)]
                        er[pl.ds(off, lanes)] = e
                        s[v % 4] = s[v % 4] + e
                        q[v % 4] = q[v % 4] + e * e
                    mean = jnp.sum((s[0] + s[1]) + (s[2] + s[3])) * (1.0 / d)
                    var = jnp.sum((q[0] + q[1]) + (q[2] + q[3])) * (1.0 / d) - mean * mean
                    mv = jnp.full((lanes,), mean)
                    rstd = _rsqrt_newton(jnp.full((lanes,), var + _EPS))

                    for v in range(nvec):
                        off = v * lanes
                        e = er[pl.ds(off, lanes)]
                        er[pl.ds(off, lanes)] = (
                            (e - mv) * rstd * gbuf[pl.ds(off, lanes)]
                            + bbuf[pl.ds(off, lanes)]
                        )
                    return 0

                lax.fori_loop(0, r, row_body, 0)
                pltpu.sync_copy(ebuf, out_hbm.at[b, pl.ds(spos, r)])
            return 0

        lax.fori_loop(0, nchunk, chunk_body, 0)

    return k(input_feat, pos_table, ln_gamma, ln_beta)


def _tc_layernorm(input_feat, pos_table, ln_gamma, ln_beta):
    bsz, seq, d = input_feat.shape
    n_s = seq // _S_BLK
    grid = (n_s, bsz)
    return pl.pallas_call(
        _ln_body,
        grid=grid,
        in_specs=[
            pl.BlockSpec((1, _S_BLK, d), lambda i, j: (j, i, 0)),
            pl.BlockSpec((_S_BLK, d), lambda i, j: (i, 0)),
            pl.BlockSpec((d,), lambda i, j: (0,)),
            pl.BlockSpec((d,), lambda i, j: (0,)),
        ],
        out_specs=pl.BlockSpec((1, _S_BLK, d), lambda i, j: (j, i, 0)),
        out_shape=jax.ShapeDtypeStruct((bsz, seq, d), input_feat.dtype),
        compiler_params=pltpu.CompilerParams(
            dimension_semantics=("arbitrary", "arbitrary"),
        ),
    )(input_feat, pos_table, ln_gamma, ln_beta)


def kernel(input_feat, pos_table, ln_gamma, ln_beta):
    return _sc_layernorm(input_feat, pos_table, ln_gamma, ln_beta)


# speedup vs baseline: 37.9824x; 9.5604x over previous
import functools

import jax
import jax.numpy as jnp
from jax import lax
from jax.experimental import pallas as pl
from jax.experimental.pallas import tpu as pltpu
from jax.experimental.pallas import tpu_sc as plsc

_EPS = 1e-5
_S_BLK = 2048


def _ln_body(x_ref, pos_ref, g_ref, b_ref, o_ref):
    x = x_ref[...]
    p = pos_ref[...]
    e = x + p[None, :, :]
    mean = jnp.mean(e, axis=-1, keepdims=True)
    c = e - mean
    var = jnp.mean(c * c, axis=-1, keepdims=True)
    o_ref[...] = c * jax.lax.rsqrt(var + _EPS) * g_ref[...] + b_ref[...]


def _rsqrt_newton(v):
    i = plsc.bitcast(v, jnp.int32)
    i = jnp.int32(0x5F3759DF) - (i >> 1)
    y = plsc.bitcast(i, jnp.float32)
    for _ in range(3):
        y = y * (1.5 - 0.5 * v * y * y)
    return y


def _sc_layernorm(input_feat, pos_table, ln_gamma, ln_beta):
    bsz, seq, d = input_feat.shape
    info = plsc.get_sparse_core_info()
    nw = info.num_cores * info.num_subcores
    lanes = info.num_lanes
    span = seq // nw
    r = lanes
    nchunk = span // r
    nvec = d // lanes
    unroll = 8
    mesh = plsc.VectorSubcoreMesh(core_axis_name="c", subcore_axis_name="s")

    @functools.partial(
        pl.kernel,
        mesh=mesh,
        out_type=jax.ShapeDtypeStruct((bsz, seq, d), jnp.float32),
        scratch_types=[
            pltpu.VMEM((r, d), jnp.float32),
            pltpu.VMEM((r, d), jnp.float32),
            pltpu.VMEM((r, d), jnp.float32),
            pltpu.VMEM((d,), jnp.float32),
            pltpu.VMEM((d,), jnp.float32),
        ],
        compiler_params=pltpu.CompilerParams(
            use_tc_tiling_on_sc=False, needs_layout_passes=False
        ),
    )
    def k(x_hbm, pos_hbm, g_hbm, b_hbm, out_hbm, xbuf, posbuf, ebuf, gbuf, bbuf):
        wid = lax.axis_index("s") * info.num_cores + lax.axis_index("c")
        s0 = wid * span
        pltpu.sync_copy(g_hbm, gbuf)
        pltpu.sync_copy(b_hbm, bbuf)

        def chunk_body(ci, _):
            spos = s0 + ci * r
            pltpu.sync_copy(pos_hbm.at[pl.ds(spos, r)], posbuf)
            for b in range(bsz):
                pltpu.sync_copy(x_hbm.at[b, pl.ds(spos, r)], xbuf)

                def row_body(rw, _):
                    xr, pr, er = xbuf.at[rw], posbuf.at[rw], ebuf.at[rw]
                    zero = jnp.zeros((lanes,), jnp.float32)
                    s = [zero] * 4
                    q = [zero] * 4
                    for v in range(nvec):
                        off = v * lanes
                        e = xr[pl.ds(off, lanes)] + pr[pl.ds(off, lanes)]
                        er[pl.ds(off, lanes)] = e
                        s[v % 4] = s[v % 4] + e
                        q[v % 4] = q[v % 4] + e * e
                    mean = jnp.sum((s[0] + s[1]) + (s[2] + s[3])) * (1.0 / d)
                    var = jnp.sum((q[0] + q[1]) + (q[2] + q[3])) * (1.0 / d) - mean * mean
                    mv = jnp.full((lanes,), mean)
                    rstd = _rsqrt_newton(jnp.full((lanes,), var + _EPS))

                    for v in range(nvec):
                        off = v * lanes
                        e = er[pl.ds(off, lanes)]
                        er[pl.ds(off, lanes)] = (
                            (e - mv) * rstd * gbuf[pl.ds(off, lanes)]
                            + bbuf[pl.ds(off, lanes)]
                        )
                    return 0

                lax.fori_loop(0, r, row_body, 0)
                pltpu.sync_copy(ebuf, out_hbm.at[b, pl.ds(spos, r)])
            return 0

        lax.fori_loop(0, nchunk, chunk_body, 0)

    return k(input_feat, pos_table, ln_gamma, ln_beta)


def _tc_layernorm(input_feat, pos_table, ln_gamma, ln_beta):
    bsz, seq, d = input_feat.shape
    n_s = seq // _S_BLK
    grid = (n_s, bsz)
    return pl.pallas_call(
        _ln_body,
        grid=grid,
        in_specs=[
            pl.BlockSpec((1, _S_BLK, d), lambda i, j: (j, i, 0)),
            pl.BlockSpec((_S_BLK, d), lambda i, j: (i, 0)),
            pl.BlockSpec((d,), lambda i, j: (0,)),
            pl.BlockSpec((d,), lambda i, j: (0,)),
        ],
        out_specs=pl.BlockSpec((1, _S_BLK, d), lambda i, j: (j, i, 0)),
        out_shape=jax.ShapeDtypeStruct((bsz, seq, d), input_feat.dtype),
        compiler_params=pltpu.CompilerParams(
            dimension_semantics=("arbitrary", "arbitrary"),
            vmem_limit_bytes=128 * 1024 * 1024,
        ),
    )(input_feat, pos_table, ln_gamma, ln_beta)


def kernel(input_feat, pos_table, ln_gamma, ln_beta):
    return _tc_layernorm(input_feat, pos_table, ln_gamma, ln_beta)
